# Initial kernel scaffold; baseline (speedup 1.0000x reference)
import jax
import jax.numpy as jnp
from jax.experimental import pallas as pl


def kernel(queries, keys, values, attn_mask):
    raise NotImplementedError("write your pallas kernel here")


# speedup vs baseline: 3.0618x; 3.0618x over previous
import jax
import jax.numpy as jnp
import numpy as np
from jax.experimental import pallas as pl
from jax.experimental.pallas import tpu as pltpu

F32 = jnp.float32
L = 2048
D = 64
U = 40
NEG = -3e38
KC = 256
NKC = L // KC
QT = 128
NQT = L // QT


def _dot_tt(a, b):
    return jax.lax.dot_general(a, b, (((1,), (1,)), ((), ())),
                               preferred_element_type=F32)


def _w_build_kernel(idx_ref, w_ref):
    idx = idx_ref[:, :]
    for c in range(NKC):
        iota = jax.lax.broadcasted_iota(jnp.int32, (KC, QT), 0) + c * KC
        acc = jnp.zeros((KC, QT), F32)
        for j in range(U):
            acc += (iota == idx[j:j + 1, :]).astype(F32)
        w_ref[c * KC:(c + 1) * KC, :] = acc.astype(jnp.int8)


def _attn_kernel(q_ref, k_ref, v_ref, w_ref, out_ref, m_ref, qsel_ref, s_ref):

    for t in range(NQT):
        qt = q_ref[0, t * QT:(t + 1) * QT, :]
        mmax = jnp.full((1, QT), NEG, F32)
        msum = jnp.zeros((1, QT), F32)
        for c in range(NKC):
            ks = k_ref[0, c * KC:(c + 1) * KC, :]
            st = _dot_tt(ks, qt)
            wt = w_ref[c * KC:(c + 1) * KC, t * QT:(t + 1) * QT].astype(F32)
            mmax = jnp.maximum(
                mmax, jnp.max(jnp.where(wt > 0, st, NEG), axis=0, keepdims=True))
            msum = msum + jnp.sum(st * wt, axis=0, keepdims=True)
        m_ref[t:t + 1, :] = mmax - msum * (1.0 / L)

    m = m_ref[:, :]
    flat = (jax.lax.broadcasted_iota(jnp.int32, (NQT, QT), 0) * QT
            + jax.lax.broadcasted_iota(jnp.int32, (NQT, QT), 1))
    sel = []
    for i in range(U):
        vmax = jnp.max(m)
        qi = jnp.min(jnp.where(m == vmax, flat, jnp.int32(1 << 30)))
        sel.append(qi)
        m = jnp.where(flat == qi, NEG, m)

    for i in range(U):
        qsel_ref[i:i + 1, :] = q_ref[0, pl.ds(sel[i], 1), :]
    qsel = qsel_ref[:, :]

    scale = F32(1.0 / np.sqrt(D))
    rowmax = jnp.full((U, 1), NEG, F32)
    for c in range(NKC):
        ks = k_ref[0, c * KC:(c + 1) * KC, :]
        sc = _dot_tt(qsel, ks) * scale
        s_ref[:, c * KC:(c + 1) * KC] = sc
        rowmax = jnp.maximum(rowmax, jnp.max(sc, axis=1, keepdims=True))
    rowsum = jnp.zeros((U, 1), F32)
    upd = jnp.zeros((U, D), F32)
    for c in range(NKC):
        p = jnp.exp(s_ref[:, c * KC:(c + 1) * KC] - rowmax)
        rowsum = rowsum + jnp.sum(p, axis=1, keepdims=True)
        upd = upd + jax.lax.dot_general(
            p, v_ref[0, c * KC:(c + 1) * KC, :], (((1,), (0,)), ((), ())),
            preferred_element_type=F32)
    upd = upd / rowsum

    acc = jnp.zeros((1, D), F32)
    for c in range(NKC):
        acc = acc + jnp.sum(v_ref[0, c * KC:(c + 1) * KC, :], axis=0,
                            keepdims=True)
    meanv = acc * (1.0 / L)
    for c in range(NKC):
        out_ref[0, c * KC:(c + 1) * KC, :] = jnp.broadcast_to(meanv, (KC, D))
    for i in range(U):
        out_ref[0, pl.ds(sel[i], 1), :] = upd[i:i + 1, :]


def kernel(queries, keys, values, attn_mask):
    B, Lq, H, Dd = queries.shape
    qh = jnp.transpose(queries.reshape(Lq, H, Dd), (1, 0, 2))
    kh = jnp.transpose(keys.reshape(Lq, H, Dd), (1, 0, 2))
    vh = jnp.transpose(values.reshape(Lq, H, Dd), (1, 0, 2))

    idx = jax.random.randint(jax.random.key(42), (Lq, U), 0, Lq)
    idx_t = idx.T

    w = pl.pallas_call(
        _w_build_kernel,
        grid=(NQT,),
        in_specs=[pl.BlockSpec((U, QT), lambda t: (0, t))],
        out_specs=pl.BlockSpec((L, QT), lambda t: (0, t)),
        out_shape=jax.ShapeDtypeStruct((L, L), jnp.int8),
    )(idx_t)

    ctx = pl.pallas_call(
        _attn_kernel,
        grid=(H,),
        in_specs=[
            pl.BlockSpec((1, L, D), lambda h: (h, 0, 0)),
            pl.BlockSpec((1, L, D), lambda h: (h, 0, 0)),
            pl.BlockSpec((1, L, D), lambda h: (h, 0, 0)),
            pl.BlockSpec((L, L), lambda h: (0, 0)),
        ],
        out_specs=pl.BlockSpec((1, L, D), lambda h: (h, 0, 0)),
        out_shape=jax.ShapeDtypeStruct((H, L, D), F32),
        scratch_shapes=[
            pltpu.VMEM((NQT, QT), F32),
            pltpu.VMEM((U, D), F32),
            pltpu.VMEM((U, L), F32),
        ],
    )(qh, kh, vh, w)

    return jnp.transpose(ctx, (1, 0, 2)).reshape(B, Lq, H, Dd)


# speedup vs baseline: 5.1143x; 1.6704x over previous
import jax
import jax.numpy as jnp
import numpy as np
from jax.experimental import pallas as pl
from jax.experimental.pallas import tpu as pltpu

F32 = jnp.float32
L = 2048
D = 64
H = 16
U = 40
NEG = -3e38
KC = 256
NKC = L // KC
QT = 128
NQT = L // QT


def _dot_tt(a, b):
    return jax.lax.dot_general(a, b, (((1,), (1,)), ((), ())),
                               preferred_element_type=F32)


def _w_build_kernel(idx_ref, wmask_ref, wcnt_ref):
    idx = idx_ref[:, :]
    for c in range(NKC):
        iota = jax.lax.broadcasted_iota(jnp.int32, (KC, QT), 0) + c * KC
        acc = jnp.zeros((KC, QT), F32)
        for j in range(U):
            acc += (iota == idx[j:j + 1, :]).astype(F32)
        wcnt_ref[c * KC:(c + 1) * KC, :] = acc
        wmask_ref[c * KC:(c + 1) * KC, :] = jnp.where(acc > 0, F32(0), NEG)


def _m_kernel(q_ref, k_ref, wmask_ref, wcnt_ref, m_ref):
    for t in range(NQT):
        qt = q_ref[0, t * QT:(t + 1) * QT, :]
        mmax = jnp.full((1, QT), NEG, F32)
        msum = jnp.zeros((1, QT), F32)
        for c in range(NKC):
            ks = k_ref[0, c * KC:(c + 1) * KC, :]
            st = _dot_tt(ks, qt)
            wm = wmask_ref[c * KC:(c + 1) * KC, t * QT:(t + 1) * QT]
            wc = wcnt_ref[c * KC:(c + 1) * KC, t * QT:(t + 1) * QT]
            mmax = jnp.maximum(mmax, jnp.max(st + wm, axis=0, keepdims=True))
            msum = msum + jnp.sum(st * wc, axis=0, keepdims=True)
        m_ref[0, :, t * QT:(t + 1) * QT] = mmax - msum * (1.0 / L)


def _topk_kernel(m_ref, sel_ref):
    m = m_ref[:, 0, :]
    col = jax.lax.broadcasted_iota(jnp.int32, (H, L), 1)
    sel = jnp.zeros((H, 128), jnp.int32)
    lane = jax.lax.broadcasted_iota(jnp.int32, (H, 128), 1)
    for i in range(U):
        rowmax = jnp.max(m, axis=1, keepdims=True)
        qidx = jnp.min(jnp.where(m == rowmax, col, jnp.int32(1 << 30)),
                       axis=1, keepdims=True)
        sel = jnp.where(lane == i, jnp.broadcast_to(qidx, (H, 128)), sel)
        m = jnp.where(col == qidx, NEG, m)
    sel_ref[:, :] = sel


def _attn_kernel(sel_ref, q_ref, k_ref, v_ref, out_ref, qsel_ref, s_ref):
    h = pl.program_id(0)
    idxs = [sel_ref[h, i] for i in range(U)]

    for i in range(U):
        qsel_ref[i:i + 1, :] = q_ref[0, pl.ds(idxs[i], 1), :]
    qsel = qsel_ref[:, :]

    scale = F32(1.0 / np.sqrt(D))
    rowmax = jnp.full((U, 1), NEG, F32)
    for c in range(NKC):
        ks = k_ref[0, c * KC:(c + 1) * KC, :]
        sc = _dot_tt(qsel, ks) * scale
        s_ref[:, c * KC:(c + 1) * KC] = sc
        rowmax = jnp.maximum(rowmax, jnp.max(sc, axis=1, keepdims=True))
    rowsum = jnp.zeros((U, 1), F32)
    upd = jnp.zeros((U, D), F32)
    for c in range(NKC):
        p = jnp.exp(s_ref[:, c * KC:(c + 1) * KC] - rowmax)
        rowsum = rowsum + jnp.sum(p, axis=1, keepdims=True)
        upd = upd + jax.lax.dot_general(
            p, v_ref[0, c * KC:(c + 1) * KC, :], (((1,), (0,)), ((), ())),
            preferred_element_type=F32)
    upd = upd / rowsum

    acc = jnp.zeros((1, D), F32)
    for c in range(NKC):
        acc = acc + jnp.sum(v_ref[0, c * KC:(c + 1) * KC, :], axis=0,
                            keepdims=True)
    meanv = acc * (1.0 / L)
    for c in range(NKC):
        out_ref[0, c * KC:(c + 1) * KC, :] = jnp.broadcast_to(meanv, (KC, D))
    for i in range(U):
        out_ref[0, pl.ds(idxs[i], 1), :] = upd[i:i + 1, :]


def kernel(queries, keys, values, attn_mask):
    B, Lq, Hh, Dd = queries.shape
    qh = jnp.transpose(queries.reshape(Lq, Hh, Dd), (1, 0, 2))
    kh = jnp.transpose(keys.reshape(Lq, Hh, Dd), (1, 0, 2))
    vh = jnp.transpose(values.reshape(Lq, Hh, Dd), (1, 0, 2))

    idx = jax.random.randint(jax.random.key(42), (Lq, U), 0, Lq)
    idx_t = idx.T

    wmask, wcnt = pl.pallas_call(
        _w_build_kernel,
        grid=(NQT,),
        in_specs=[pl.BlockSpec((U, QT), lambda t: (0, t))],
        out_specs=[pl.BlockSpec((L, QT), lambda t: (0, t)),
                   pl.BlockSpec((L, QT), lambda t: (0, t))],
        out_shape=[jax.ShapeDtypeStruct((L, L), F32),
                   jax.ShapeDtypeStruct((L, L), F32)],
    )(idx_t)

    m = pl.pallas_call(
        _m_kernel,
        grid=(Hh,),
        in_specs=[
            pl.BlockSpec((1, L, D), lambda h: (h, 0, 0)),
            pl.BlockSpec((1, L, D), lambda h: (h, 0, 0)),
            pl.BlockSpec((L, L), lambda h: (0, 0)),
            pl.BlockSpec((L, L), lambda h: (0, 0)),
        ],
        out_specs=pl.BlockSpec((1, 1, L), lambda h: (h, 0, 0)),
        out_shape=jax.ShapeDtypeStruct((Hh, 1, L), F32),
    )(qh, kh, wmask, wcnt)

    sel = pl.pallas_call(
        _topk_kernel,
        out_shape=jax.ShapeDtypeStruct((Hh, 128), jnp.int32),
    )(m)

    ctx = pl.pallas_call(
        _attn_kernel,
        grid=(Hh,),
        in_specs=[
            pl.BlockSpec(memory_space=pltpu.SMEM),
            pl.BlockSpec((1, L, D), lambda h: (h, 0, 0)),
            pl.BlockSpec((1, L, D), lambda h: (h, 0, 0)),
            pl.BlockSpec((1, L, D), lambda h: (h, 0, 0)),
        ],
        out_specs=pl.BlockSpec((1, L, D), lambda h: (h, 0, 0)),
        out_shape=jax.ShapeDtypeStruct((Hh, L, D), F32),
        scratch_shapes=[
            pltpu.VMEM((U, D), F32),
            pltpu.VMEM((U, L), F32),
        ],
    )(sel, qh, kh, vh)

    return jnp.transpose(ctx, (1, 0, 2)).reshape(B, Lq, Hh, Dd)
